# baseline (speedup 1.0000x reference)
import jax
import jax.numpy as jnp
from jax.experimental import pallas as pl
from jax.experimental.pallas import tpu as pltpu

BN = 128


def _fused_net_kernel(one_ref, x_ref, t1_ref, w2a_ref, b2_ref, wf1_ref,
                      bf1_ref, wf2_ref, bf2_ref, out_ref,
                      x3_ref, h1_ref, f_ref):
    bn = x_ref.shape[1]

    for kh in range(3):
        x3_ref[:, :, kh * 28:(kh + 1) * 28] = x_ref[kh:kh + 26, :, :]
    x3_ref[:, :, 84:96] = jnp.ones((26, bn, 12), jnp.bfloat16)
    z = jnp.dot(x3_ref[...].reshape(26 * bn, 96), t1_ref[...],
                preferred_element_type=jnp.float32)
    h1v = jnp.maximum(z, 0.0).astype(jnp.bfloat16)
    h1v = h1v.reshape(13, 2, bn, 896)
    h1_ref[0] = h1v[:, 0]
    h1_ref[1] = h1v[:, 1]

    one = one_ref[0]
    for owg in range(6):
        accs = []
        for par in range(2):
            acc = None
            for kh in range(3):
                q, s = (par + kh) % 2, (par + kh) // 2
                lhs = h1_ref[q, pl.ds(s, 12), :,
                             owg * 128:owg * 128 + 256].reshape(12 * bn, 256)
                d = jnp.dot(lhs, w2a_ref[kh],
                            preferred_element_type=jnp.float32)
                acc = d * one if acc is None else acc + d
            accs.append(acc)
        ph = jnp.maximum(accs[0], accs[1])
        pooled = jnp.maximum(ph[:, 0:128], ph[:, 128:256])
        pooled = jnp.maximum(pooled + b2_ref[...], 0.0)
        f_ref[:, :, owg * 128:(owg + 1) * 128] = (
            pooled.astype(jnp.bfloat16).reshape(12, bn, 128))

    a1 = jnp.dot(f_ref[0], wf1_ref[0], preferred_element_type=jnp.float32)
    for oh in range(1, 12):
        a1 = a1 + jnp.dot(f_ref[oh], wf1_ref[oh],
                          preferred_element_type=jnp.float32)
    a1 = jnp.maximum(a1 + bf1_ref[...], 0.0)
    logits = jnp.dot(a1, wf2_ref[...],
                     preferred_element_type=jnp.float32) + bf2_ref[...]
    mx = jnp.max(logits, axis=-1, keepdims=True)
    e = jnp.exp(logits - mx)
    lse = jnp.log(jnp.sum(e, axis=-1, keepdims=True)) + mx
    out_ref[...] = logits - lse


def kernel(x, w1, b1, w2, b2, wf1, bf1, wf2, bf2):
    N = x.shape[0]
    assert x.shape[1:] == (1, 28, 28)
    Np = ((N + BN - 1) // BN) * BN
    xs = x[:, 0].astype(jnp.float32)
    if Np != N:
        xs = jnp.pad(xs, ((0, Np - N), (0, 0), (0, 0)))
    xt = jnp.transpose(xs, (1, 0, 2)).astype(jnp.bfloat16)

    w1r = jnp.transpose(w1[:, 0], (1, 2, 0))
    rows = []
    for kh in range(3):
        t = jnp.zeros((28, 26, 32), jnp.float32)
        for kw in range(3):
            E = jnp.eye(28, 26, k=-kw, dtype=jnp.float32)
            t = t + E[:, :, None] * w1r[kh, kw][None, None, :]
        rows.append(t.reshape(28, 832))
    T1 = jnp.pad(jnp.concatenate(rows, 0), ((0, 0), (0, 64)))
    b1row = jnp.pad(jnp.tile(b1.reshape(1, 32), (1, 26)), ((0, 0), (0, 64)))
    T1 = jnp.concatenate([T1, b1row, jnp.zeros((11, 896), jnp.float32)], 0)
    T1 = T1.astype(jnp.bfloat16)

    w2r = jnp.transpose(w2, (2, 3, 1, 0))
    mats = []
    for kh in range(3):
        a = jnp.zeros((8, 32, 4, 64), jnp.float32)
        for kw in range(3):
            E = jnp.eye(8, 4, k=-kw, dtype=jnp.float32)
            a = a + E[:, None, :, None] * w2r[kh, kw][:, None, :][None]
        a = a[:, :, jnp.array([0, 2, 1, 3]), :]
        mats.append(a.reshape(256, 256))
    W2a = jnp.stack(mats, 0).astype(jnp.bfloat16)

    b2h = jnp.tile(b2.reshape(1, 64), (1, 2))
    wf1r = jnp.transpose(wf1.reshape(128, 64, 12, 12),
                         (2, 3, 1, 0)).reshape(12, 768, 128).astype(jnp.bfloat16)
    bf1r = bf1.reshape(1, 128)
    wf2r = wf2.T
    bf2r = bf2.reshape(1, 10)

    out = pl.pallas_call(
        _fused_net_kernel,
        out_shape=jax.ShapeDtypeStruct((Np, 10), jnp.float32),
        grid=(Np // BN,),
        in_specs=[
            pl.BlockSpec(memory_space=pltpu.SMEM),
            pl.BlockSpec((28, BN, 28), lambda n: (0, n, 0)),
            pl.BlockSpec((96, 896), lambda n: (0, 0)),
            pl.BlockSpec((3, 256, 256), lambda n: (0, 0, 0)),
            pl.BlockSpec((1, 128), lambda n: (0, 0)),
            pl.BlockSpec((12, 768, 128), lambda n: (0, 0, 0)),
            pl.BlockSpec((1, 128), lambda n: (0, 0)),
            pl.BlockSpec((128, 10), lambda n: (0, 0)),
            pl.BlockSpec((1, 10), lambda n: (0, 0)),
        ],
        out_specs=pl.BlockSpec((BN, 10), lambda n: (n, 0)),
        scratch_shapes=[
            pltpu.VMEM((26, BN, 96), jnp.bfloat16),
            pltpu.VMEM((2, 13, BN, 896), jnp.bfloat16),
            pltpu.VMEM((12, BN, 768), jnp.bfloat16),
        ],
        compiler_params=pltpu.CompilerParams(
            dimension_semantics=("parallel",),
            vmem_limit_bytes=56 * 1024 * 1024),
    )(jnp.ones((1,), jnp.float32), xt, T1, W2a, b2h, wf1r, bf1r, wf2r, bf2r)
    return out[:N]


# speedup vs baseline: 1.0029x; 1.0029x over previous
import jax
import jax.numpy as jnp
from jax.experimental import pallas as pl
from jax.experimental.pallas import tpu as pltpu

BN = 128


def _fused_net_kernel(x_ref, t1_ref, w2a_ref, b2_ref, wf1_ref,
                      bf1_ref, wf2_ref, bf2_ref, out_ref,
                      x3_ref, h1_ref, f_ref):
    bn = x_ref.shape[1]

    for kh in range(3):
        x3_ref[:, :, kh * 28:(kh + 1) * 28] = x_ref[kh:kh + 26, :, :]
    x3_ref[:, :, 84:96] = jnp.ones((26, bn, 12), jnp.bfloat16)
    z = jnp.dot(x3_ref[...].reshape(26 * bn, 96), t1_ref[...],
                preferred_element_type=jnp.float32)
    h1v = jnp.maximum(z, 0.0).astype(jnp.bfloat16)
    h1v = h1v.reshape(13, 2, bn, 896)
    h1_ref[0] = h1v[:, 0]
    h1_ref[1] = h1v[:, 1]

    for owg in range(6):
        accs = []
        for par in range(2):
            acc = None
            for kh in range(3):
                q, s = (par + kh) % 2, (par + kh) // 2
                lhs = h1_ref[q, pl.ds(s, 12), :,
                             owg * 128:owg * 128 + 256].reshape(12 * bn, 256)
                d = jnp.dot(lhs, w2a_ref[kh],
                            preferred_element_type=jnp.float32)
                acc = d if acc is None else acc + d
            accs.append(acc)
        ph = jnp.maximum(accs[0], accs[1])
        pooled = jnp.maximum(ph[:, 0:128], ph[:, 128:256])
        pooled = jnp.maximum(pooled + b2_ref[...], 0.0)
        f_ref[:, :, owg * 128:(owg + 1) * 128] = (
            pooled.astype(jnp.bfloat16).reshape(12, bn, 128))

    a1 = jnp.dot(f_ref[0], wf1_ref[0], preferred_element_type=jnp.float32)
    for oh in range(1, 12):
        a1 = a1 + jnp.dot(f_ref[oh], wf1_ref[oh],
                          preferred_element_type=jnp.float32)
    a1 = jnp.maximum(a1 + bf1_ref[...], 0.0)
    logits = jnp.dot(a1, wf2_ref[...],
                     preferred_element_type=jnp.float32) + bf2_ref[...]
    mx = jnp.max(logits, axis=-1, keepdims=True)
    e = jnp.exp(logits - mx)
    lse = jnp.log(jnp.sum(e, axis=-1, keepdims=True)) + mx
    out_ref[...] = logits - lse


def kernel(x, w1, b1, w2, b2, wf1, bf1, wf2, bf2):
    N = x.shape[0]
    assert x.shape[1:] == (1, 28, 28)
    Np = ((N + BN - 1) // BN) * BN
    xs = x[:, 0].astype(jnp.float32)
    if Np != N:
        xs = jnp.pad(xs, ((0, Np - N), (0, 0), (0, 0)))
    xt = jnp.transpose(xs, (1, 0, 2)).astype(jnp.bfloat16)

    w1r = jnp.transpose(w1[:, 0], (1, 2, 0))
    rows = []
    for kh in range(3):
        t = jnp.zeros((28, 26, 32), jnp.float32)
        for kw in range(3):
            E = jnp.eye(28, 26, k=-kw, dtype=jnp.float32)
            t = t + E[:, :, None] * w1r[kh, kw][None, None, :]
        rows.append(t.reshape(28, 832))
    T1 = jnp.pad(jnp.concatenate(rows, 0), ((0, 0), (0, 64)))
    b1row = jnp.pad(jnp.tile(b1.reshape(1, 32), (1, 26)), ((0, 0), (0, 64)))
    T1 = jnp.concatenate([T1, b1row, jnp.zeros((11, 896), jnp.float32)], 0)
    T1 = T1.astype(jnp.bfloat16)

    w2r = jnp.transpose(w2, (2, 3, 1, 0))
    mats = []
    for kh in range(3):
        a = jnp.zeros((8, 32, 4, 64), jnp.float32)
        for kw in range(3):
            E = jnp.eye(8, 4, k=-kw, dtype=jnp.float32)
            a = a + E[:, None, :, None] * w2r[kh, kw][:, None, :][None]
        a = a[:, :, jnp.array([0, 2, 1, 3]), :]
        mats.append(a.reshape(256, 256))
    W2a = jnp.stack(mats, 0).astype(jnp.bfloat16)

    b2h = jnp.tile(b2.reshape(1, 64), (1, 2))
    wf1r = jnp.transpose(wf1.reshape(128, 64, 12, 12),
                         (2, 3, 1, 0)).reshape(12, 768, 128).astype(jnp.bfloat16)
    bf1r = bf1.reshape(1, 128)
    wf2r = wf2.T
    bf2r = bf2.reshape(1, 10)

    out = pl.pallas_call(
        _fused_net_kernel,
        out_shape=jax.ShapeDtypeStruct((Np, 10), jnp.float32),
        grid=(Np // BN,),
        in_specs=[
            pl.BlockSpec((28, BN, 28), lambda n: (0, n, 0)),
            pl.BlockSpec((96, 896), lambda n: (0, 0)),
            pl.BlockSpec((3, 256, 256), lambda n: (0, 0, 0)),
            pl.BlockSpec((1, 128), lambda n: (0, 0)),
            pl.BlockSpec((12, 768, 128), lambda n: (0, 0, 0)),
            pl.BlockSpec((1, 128), lambda n: (0, 0)),
            pl.BlockSpec((128, 10), lambda n: (0, 0)),
            pl.BlockSpec((1, 10), lambda n: (0, 0)),
        ],
        out_specs=pl.BlockSpec((BN, 10), lambda n: (n, 0)),
        scratch_shapes=[
            pltpu.VMEM((26, BN, 96), jnp.bfloat16),
            pltpu.VMEM((2, 13, BN, 896), jnp.bfloat16),
            pltpu.VMEM((12, BN, 768), jnp.bfloat16),
        ],
        compiler_params=pltpu.CompilerParams(
            dimension_semantics=("parallel",),
            vmem_limit_bytes=56 * 1024 * 1024),
    )(xt, T1, W2a, b2h, wf1r, bf1r, wf2r, bf2r)
    return out[:N]
